# baseline (speedup 1.0000x reference)
import functools

import jax
import jax.numpy as jnp
from jax.experimental import pallas as pl

_BLOCK_M = 2048
_NSTREAM = 2


def _heads_body(na, ncls, nreg, bm, *refs):
    x_refs = refs[:_NSTREAM]
    w_ref, b_ref = refs[_NSTREAM:_NSTREAM + 2]
    conf_ref, cls_ref, reg_ref = refs[_NSTREAM + 2:]
    w = w_ref[...]
    b = b_ref[...]
    for k in range(_NSTREAM):
        yt = jax.lax.dot_general(
            w, x_refs[k][...],
            dimension_numbers=(((0,), (1,)), ((), ())),
            preferred_element_type=jnp.float32,
        ) + b
        cols = pl.ds(k * bm, bm)
        conf_ref[:, cols] = yt[:na, :]
        cls_ref[:, cols] = yt[na:na + ncls, :]
        reg_ref[:, cols] = yt[na + ncls:, :]


def kernel(hidden_states, W_conf, b_conf, W_cls, b_cls, W_reg, b_reg):
    B, S, H = hidden_states.shape
    M = B * S
    na = W_conf.shape[1]
    ncls = W_cls.shape[1]
    nreg = W_reg.shape[1]
    nl = ncls // na
    n_all = na + ncls + nreg

    x = hidden_states.reshape(M, H)
    bm = min(_BLOCK_M, M // _NSTREAM)
    group = bm * _NSTREAM

    w_all = jnp.concatenate([W_conf, W_cls, W_reg], axis=1)
    b_all = jnp.concatenate([b_conf, b_cls, b_reg]).reshape(n_all, 1)

    body = functools.partial(_heads_body, na, ncls, nreg, bm)

    def x_spec(k):
        return pl.BlockSpec((bm, H), lambda i, k=k: (i * _NSTREAM + k, 0))

    conf_t, cls_t, reg_t = pl.pallas_call(
        body,
        grid=(M // group,),
        in_specs=[x_spec(k) for k in range(_NSTREAM)] + [
            pl.BlockSpec((H, n_all), lambda i: (0, 0)),
            pl.BlockSpec((n_all, 1), lambda i: (0, 0)),
        ],
        out_specs=[
            pl.BlockSpec((na, group), lambda i: (0, i)),
            pl.BlockSpec((ncls, group), lambda i: (0, i)),
            pl.BlockSpec((nreg, group), lambda i: (0, i)),
        ],
        out_shape=[
            jax.ShapeDtypeStruct((na, M), jnp.float32),
            jax.ShapeDtypeStruct((ncls, M), jnp.float32),
            jax.ShapeDtypeStruct((nreg, M), jnp.float32),
        ],
    )(*([x] * _NSTREAM), w_all, b_all)

    return (
        conf_t.T.reshape(B, S, na),
        cls_t.T.reshape(B, S, na, nl),
        reg_t.T.reshape(B, S, na, 2),
    )
